# baseline (speedup 1.0000x reference)
import functools

import jax
import jax.numpy as jnp
from jax import lax
from jax.experimental import pallas as pl
from jax.experimental.pallas import tpu as pltpu
from jax.experimental.pallas import tpu_sc as plsc

DICT_SIZE = 100000
EMB = 128
BATCH = 16384

NUM_CORES = 2
NUM_SUBCORES = 16
NUM_WORKERS = NUM_CORES * NUM_SUBCORES
BPW = BATCH // NUM_WORKERS
CHUNK = 128
NCHUNKS = BPW // CHUNK
GROUPS = CHUNK // 16
LANES = 16
VPR = EMB // LANES

_mesh = plsc.VectorSubcoreMesh(core_axis_name="c", subcore_axis_name="s")


def _tree_add(parts):
    while len(parts) > 1:
        parts = [parts[i] + parts[i + 1]
                 for i in range(0, len(parts) - 1, 2)] \
            + ([parts[-1]] if len(parts) % 2 else [])
    return parts[0]


@functools.partial(
    pl.kernel,
    out_type=jax.ShapeDtypeStruct((BATCH,), jnp.float32),
    mesh=_mesh,
    compiler_params=pltpu.CompilerParams(needs_layout_passes=False),
    scratch_types=[
        pltpu.VMEM((BPW,), jnp.int32),
        pltpu.VMEM((BPW,), jnp.int32),
        pltpu.VMEM((2, CHUNK, EMB), jnp.float32),
        pltpu.VMEM((2, CHUNK, EMB), jnp.float32),
        pltpu.VMEM((2 * LANES * LANES,), jnp.float32),
        pltpu.VMEM((BPW,), jnp.float32),
        pltpu.SemaphoreType.DMA,
        pltpu.SemaphoreType.DMA,
    ],
)
def _word_sim(idx_hbm, tgt_hbm, table_hbm, out_hbm,
              idx_v, tgt_v, xs_v, ys_v, tp_v, out_v, sem0, sem1):
    wid = lax.axis_index("s") * NUM_CORES + lax.axis_index("c")
    base = wid * BPW
    cp_i = pltpu.async_copy(idx_hbm.at[pl.ds(base, BPW)], idx_v, sem0)
    cp_t = pltpu.async_copy(tgt_hbm.at[pl.ds(base, BPW)], tgt_v, sem1)
    cp_i.wait()
    cp_t.wait()

    lane = lax.broadcasted_iota(jnp.int32, (LANES,), 0)
    sems = (sem0, sem1)

    def fire(c):
        buf = c % 2
        cpx = pltpu.async_copy(
            table_hbm.at[idx_v.at[pl.ds(c * CHUNK, CHUNK)]],
            xs_v.at[buf], sems[buf])
        cpy = pltpu.async_copy(
            table_hbm.at[tgt_v.at[pl.ds(c * CHUNK, CHUNK)]],
            ys_v.at[buf], sems[buf])
        return cpx, cpy

    pending = fire(0)
    for c in range(NCHUNKS):
        nxt = fire(c + 1) if c + 1 < NCHUNKS else None
        pending[0].wait()
        pending[1].wait()
        buf = c % 2

        def pair_body(h, carry, c=c, buf=buf):
            for half in range(2):
                row0 = (h * 2 + half) * LANES
                for r in range(LANES):
                    p = _tree_add(
                        [xs_v[buf, row0 + r, pl.ds(k * LANES, LANES)]
                         * ys_v[buf, row0 + r, pl.ds(k * LANES, LANES)]
                         for k in range(VPR)])
                    plsc.store_scatter(
                        tp_v, [lane * LANES + (half * LANES * LANES + r)], p)
            for half in range(2):
                row0 = (h * 2 + half) * LANES
                res = _tree_add(
                    [tp_v[pl.ds(half * LANES * LANES + j * LANES, LANES)]
                     for j in range(LANES)])
                out_v[pl.ds(c * CHUNK + row0, LANES)] = res
            return carry

        lax.fori_loop(0, GROUPS // 2, pair_body, 0)
        pending = nxt

    pltpu.sync_copy(out_v, out_hbm.at[pl.ds(base, BPW)])


def kernel(idx, targets, word_embs):
    return _word_sim(idx, targets, word_embs)


# speedup vs baseline: 1.2402x; 1.2402x over previous
import functools

import jax
import jax.numpy as jnp
from jax import lax
from jax.experimental import pallas as pl
from jax.experimental.pallas import tpu as pltpu
from jax.experimental.pallas import tpu_sc as plsc

DICT_SIZE = 100000
EMB = 128
BATCH = 16384

NUM_CORES = 2
NUM_SUBCORES = 16
NUM_WORKERS = NUM_CORES * NUM_SUBCORES
BPW = BATCH // NUM_WORKERS
CHUNK = 128
NCHUNKS = BPW // CHUNK
GROUPS = CHUNK // 16
LANES = 16
VPR = EMB // LANES

_mesh = plsc.VectorSubcoreMesh(core_axis_name="c", subcore_axis_name="s")


def _tree_add(parts):
    while len(parts) > 1:
        parts = [parts[i] + parts[i + 1]
                 for i in range(0, len(parts) - 1, 2)] \
            + ([parts[-1]] if len(parts) % 2 else [])
    return parts[0]


@functools.partial(
    pl.kernel,
    out_type=jax.ShapeDtypeStruct((BATCH,), jnp.float32),
    mesh=_mesh,
    compiler_params=pltpu.CompilerParams(needs_layout_passes=False),
    scratch_types=[
        pltpu.VMEM((BPW,), jnp.int32),
        pltpu.VMEM((BPW,), jnp.int32),
        pltpu.VMEM((2, CHUNK, EMB), jnp.float32),
        pltpu.VMEM((2, CHUNK, EMB), jnp.float32),
        pltpu.VMEM((2 * LANES * LANES,), jnp.float32),
        pltpu.VMEM((BPW,), jnp.float32),
        pltpu.SemaphoreType.DMA,
        pltpu.SemaphoreType.DMA,
    ],
)
def _word_sim(idx_hbm, tgt_hbm, table_hbm, out_hbm,
              idx_v, tgt_v, xs_v, ys_v, tp_v, out_v, sem0, sem1):
    wid = lax.axis_index("s") * NUM_CORES + lax.axis_index("c")
    base = wid * BPW
    cp_i = pltpu.async_copy(idx_hbm.at[pl.ds(base, BPW)], idx_v, sem0)
    cp_t = pltpu.async_copy(tgt_hbm.at[pl.ds(base, BPW)], tgt_v, sem1)
    cp_i.wait()
    cp_t.wait()

    lane = lax.broadcasted_iota(jnp.int32, (LANES,), 0)
    sems = (sem0, sem1)

    def fire(c):
        buf = c % 2
        cpx = pltpu.async_copy(
            table_hbm.at[idx_v.at[pl.ds(c * CHUNK, CHUNK)]],
            xs_v.at[buf], sems[buf])
        cpy = pltpu.async_copy(
            table_hbm.at[tgt_v.at[pl.ds(c * CHUNK, CHUNK)]],
            ys_v.at[buf], sems[buf])
        return cpx, cpy

    pending = fire(0)
    for c in range(NCHUNKS):
        nxt = fire(c + 1) if c + 1 < NCHUNKS else None
        pending[0].wait()
        pending[1].wait()
        buf = c % 2

        def group_body(g, carry, c=c, buf=buf):
            row0 = g * LANES
            h = (g % 2) * (LANES * LANES)
            def row_loads(r):
                return ([xs_v[buf, row0 + r, pl.ds(k * LANES, LANES)]
                         for k in range(VPR)],
                        [ys_v[buf, row0 + r, pl.ds(k * LANES, LANES)]
                         for k in range(VPR)])

            xv, yv = row_loads(0)
            for r in range(LANES):
                nxt = row_loads(r + 1) if r + 1 < LANES else None
                p = _tree_add([xv[k] * yv[k] for k in range(VPR)])
                tp_v[pl.ds(h + r * LANES, LANES)] = p
                if nxt is not None:
                    xv, yv = nxt
            res = _tree_add(
                [plsc.load_gather(tp_v, [lane * LANES + (j + h)])
                 for j in range(LANES)])
            out_v[pl.ds(c * CHUNK + row0, LANES)] = res
            return carry

        lax.fori_loop(0, GROUPS, group_body, 0)
        pending = nxt

    pltpu.sync_copy(out_v, out_hbm.at[pl.ds(base, BPW)])


def kernel(idx, targets, word_embs):
    return _word_sim(idx, targets, word_embs)
